# baseline (speedup 1.0000x reference)
import functools

import jax
import jax.numpy as jnp
from jax import lax
from jax.experimental import pallas as pl
from jax.experimental.pallas import tpu as pltpu
from jax.experimental.pallas import tpu_sc as plsc

S, B, D = 4096, 4, 4096
N = S // 2
SBLK = 256
GRID = S // SBLK
L = 16
NV = S // L


def _gumbel_table():
    u = jax.random.uniform(jax.random.key(42), (B, S), minval=1e-20,
                           maxval=1.0)
    return -jnp.log(-jnp.log(u))


def _sortable_i32(f):
    b = jax.lax.bitcast_convert_type(f, jnp.int32)
    flip = jax.lax.shift_right_arithmetic(b, 31).astype(jnp.uint32) \
        | jnp.uint32(0x80000000)
    ku = b.astype(jnp.uint32) ^ flip
    return jax.lax.bitcast_convert_type(ku ^ jnp.uint32(0x80000000),
                                        jnp.int32)


def _tc_body(x_ref, te_ref, g_ref, w_ref, k_ref, w_acc):
    i = pl.program_id(0)

    xb = x_ref[...]
    te = te_ref[...]
    part = jnp.sum(xb * te[None, :, :], axis=-1)
    w_acc[:, pl.ds(i * SBLK, SBLK)] = part.T

    @pl.when(i == GRID - 1)
    def _():
        w = w_acc[...]
        g = g_ref[...]
        mx = jnp.max(w, axis=1, keepdims=True)
        t = mx - w
        tmx = jnp.max(t, axis=1, keepdims=True)
        p = jnp.exp(t - tmx)
        p_inv = p / jnp.sum(p, axis=1, keepdims=True)
        sc = jnp.log(p_inv + 1e-20) + g
        w_ref[...] = w
        k_ref[...] = _sortable_i32(sc)


def _radix_level(k_v, hist, shift, prev_ok_fn, rank):
    ones = jnp.ones((L,), jnp.int32)

    def zero_body(j, _):
        hist[pl.ds(j * L, L)] = jnp.zeros((L,), jnp.int32)
        return 0
    lax.fori_loop(0, 256 // L, zero_body, 0)

    def hbody(j, _):
        kv = k_v[pl.ds(j * L, L)]
        if shift == 24:
            bkt = lax.shift_right_arithmetic(kv, 24) + jnp.int32(128)
        else:
            bkt = lax.shift_right_arithmetic(kv, shift) & jnp.int32(0xFF)
        plsc.addupdate_scatter(hist, [bkt], ones, mask=prev_ok_fn(kv))
        return 0
    lax.fori_loop(0, NV, hbody, 0)

    lane = lax.iota(jnp.int32, L)

    def sbody(j, carry):
        acc, bstar, above, found = carry
        jj = jnp.int32(256 // L - 1) - j
        hv = hist[pl.ds(jj * L, L)]
        rv = lax.rev(hv, (0,))
        cum = lax.cumsum(rv, axis=0)
        m = jnp.logical_and(jnp.logical_not(found), (acc + cum) >= rank)
        fl = jnp.min(jnp.where(m, lane, jnp.int32(L)), axis=0)
        hit = fl < L
        sel = lane == fl
        above_here = jnp.sum(jnp.where(sel, acc + cum - rv, 0), axis=0)
        bstar = jnp.where(hit, jj * L + (L - 1 - fl), bstar)
        above = jnp.where(hit, above_here, above)
        found = jnp.logical_or(found, hit)
        acc = acc + jnp.sum(hv, axis=0)
        return acc, bstar, above, found
    _, bstar, above, _ = lax.fori_loop(
        0, 256 // L, sbody,
        (jnp.int32(0), jnp.int32(0), jnp.int32(0), False))
    return bstar, above, rank - above


def _sc_body(w_hbm, k_hbm, out_hbm, w_v, k_v, e_v, hist):
    wid = lax.axis_index("s") * 2 + lax.axis_index("c")

    @pl.when(wid < B)
    def _():
        base = wid * S
        pltpu.sync_copy(w_hbm.at[pl.ds(base, S)], w_v)
        pltpu.sync_copy(k_hbm.at[pl.ds(base, S)], k_v)

        b1, a1, r1 = _radix_level(
            k_v, hist, 24, lambda kv: kv == kv, jnp.int32(N))

        def ok1(kv):
            return (lax.shift_right_arithmetic(kv, 24) + 128) == b1
        b2, a2, r2 = _radix_level(k_v, hist, 16, ok1, r1)

        def ok2(kv):
            return jnp.logical_and(
                ok1(kv), (lax.shift_right_arithmetic(kv, 16) & 0xFF) == b2)
        b3, a3, r3 = _radix_level(k_v, hist, 8, ok2, r2)

        def ok3(kv):
            return jnp.logical_and(
                ok2(kv), (lax.shift_right_arithmetic(kv, 8) & 0xFF) == b3)
        b4, a4, r4 = _radix_level(k_v, hist, 0, ok3, r3)

        thr = (lax.shift_left(b1 - jnp.int32(128), 24)
               | lax.shift_left(b2, 16) | lax.shift_left(b3, 8) | b4)
        r_tie = r4

        lane = lax.iota(jnp.int32, L)

        def tbody(j, carry):
            acc, cidx = carry
            kv = k_v[pl.ds(j * L, L)]
            m = (kv == thr).astype(jnp.int32)
            cum = lax.cumsum(m, axis=0)
            sel = jnp.logical_and(m == 1, (acc + cum) == r_tie)
            idxv = jnp.where(sel, lane + j * L, jnp.int32(-1))
            cand = jnp.max(idxv, axis=0)
            cidx = jnp.maximum(cidx, cand)
            return acc + jnp.sum(m, axis=0), cidx
        _, cidx = lax.fori_loop(0, NV, tbody, (jnp.int32(0), jnp.int32(-1)))

        def mask_of(j, kv):
            idxv = lane + j * L
            return jnp.logical_or(
                kv > thr,
                jnp.logical_and(kv == thr, idxv <= cidx))

        def mbody(j, m2):
            kv = k_v[pl.ds(j * L, L)]
            wv = w_v[pl.ds(j * L, L)]
            wm = jnp.where(mask_of(j, kv), jnp.float32(-3.4e38), wv)
            return jnp.maximum(m2, jnp.max(wm, axis=0))
        m2 = lax.fori_loop(0, NV, mbody, jnp.float32(-3.4e38))

        def ebody(j, s):
            kv = k_v[pl.ds(j * L, L)]
            wv = w_v[pl.ds(j * L, L)]
            e = jnp.where(mask_of(j, kv), jnp.float32(0.0),
                          jnp.exp(wv - m2))
            e_v[pl.ds(j * L, L)] = e
            return s + jnp.sum(e, axis=0)
        ssum = lax.fori_loop(0, NV, ebody, jnp.float32(0.0))

        invv = jnp.ones((L,), jnp.float32) / jnp.broadcast_to(ssum, (L,))

        def nbody(j, _):
            e_v[pl.ds(j * L, L)] = e_v[pl.ds(j * L, L)] * invv
            return 0
        lax.fori_loop(0, NV, nbody, 0)

        pltpu.sync_copy(e_v, out_hbm.at[pl.ds(base, S)])


def _sc_probe_body(x_hbm, out_hbm, buf, acc_v):
    wid = lax.axis_index("s") * 2 + lax.axis_index("c")
    base = wid * (32 * 4096)

    def body(i, _):
        pltpu.sync_copy(x_hbm.at[pl.ds(base + i * 4096, 4096)], buf)
        acc = jnp.zeros((L,), jnp.float32)

        def rbody(j, a):
            return a + buf[pl.ds(j * L, L)]
        acc = lax.fori_loop(0, 4096 // L, rbody, acc)
        acc_v[...] = acc
        return 0
    lax.fori_loop(0, 32, body, 0)
    pltpu.sync_copy(acc_v, out_hbm.at[pl.ds(wid * L, L)])


@jax.jit
def kernel(x, te):
    te2 = te[..., 0]
    g = _gumbel_table()

    probe = functools.partial(
        pl.kernel,
        mesh=plsc.VectorSubcoreMesh(core_axis_name="c", subcore_axis_name="s"),
        out_type=jax.ShapeDtypeStruct((32 * L,), jnp.float32),
        scratch_types=[
            pltpu.VMEM((4096,), jnp.float32),
            pltpu.VMEM((L,), jnp.float32),
        ],
        compiler_params=pltpu.CompilerParams(needs_layout_passes=False),
    )(_sc_probe_body)
    dummy = probe(x.reshape(S * B * D))
    w, k = pl.pallas_call(
        _tc_body,
        grid=(GRID,),
        in_specs=[
            pl.BlockSpec((SBLK, B, D), lambda i: (i, 0, 0)),
            pl.BlockSpec((B, D), lambda i: (0, 0)),
            pl.BlockSpec((B, S), lambda i: (0, 0)),
        ],
        out_specs=[
            pl.BlockSpec((B, S), lambda i: (0, 0)),
            pl.BlockSpec((B, S), lambda i: (0, 0)),
        ],
        out_shape=[
            jax.ShapeDtypeStruct((B, S), jnp.float32),
            jax.ShapeDtypeStruct((B, S), jnp.int32),
        ],
        scratch_shapes=[pltpu.VMEM((B, S), jnp.float32)],
        compiler_params=pltpu.CompilerParams(
            dimension_semantics=("arbitrary",),
        ),
    )(x, te2, g)

    mesh = plsc.VectorSubcoreMesh(core_axis_name="c", subcore_axis_name="s")
    sc = functools.partial(
        pl.kernel, mesh=mesh,
        out_type=jax.ShapeDtypeStruct((B * S,), jnp.float32),
        scratch_types=[
            pltpu.VMEM((S,), jnp.float32),
            pltpu.VMEM((S,), jnp.int32),
            pltpu.VMEM((S,), jnp.float32),
            pltpu.VMEM((256,), jnp.int32),
        ],
        compiler_params=pltpu.CompilerParams(needs_layout_passes=False),
    )(_sc_body)
    out = sc(w.reshape(B * S), k.reshape(B * S))
    out = out + 0.0 * dummy[0]
    return out.reshape(B, S).T[..., None]


# speedup vs baseline: 2.6245x; 2.6245x over previous
import functools

import jax
import jax.numpy as jnp
from jax import lax
from jax.experimental import pallas as pl
from jax.experimental.pallas import tpu as pltpu
from jax.experimental.pallas import tpu_sc as plsc

S, B, D = 4096, 4, 4096
N = S // 2
L = 16
NSUB = 32

RS = 1280
S1 = S - RS
NR = RS // NSUB
NG = NR // 2

SBLK = 256
GRID = S1 // SBLK


def _gumbel_table():
    u = jax.random.uniform(jax.random.key(42), (B, S), minval=1e-20,
                           maxval=1.0)
    return -jnp.log(-jnp.log(u))


def _sortable_u32(f):
    b = jax.lax.bitcast_convert_type(f, jnp.int32)
    flip = jax.lax.shift_right_arithmetic(b, 31).astype(jnp.uint32) \
        | jnp.uint32(0x80000000)
    return b.astype(jnp.uint32) ^ flip


def _bf16_rne(v):
    u = lax.bitcast_convert_type(v, jnp.int32)
    r = u + jnp.int32(0x7FFF) + \
        (lax.shift_right_logical(u, jnp.int32(16)) & jnp.int32(1))
    return lax.bitcast_convert_type(r & jnp.int32(-65536), jnp.float32)


def _sc_dense_body(x_hbm, te_hbm, w2p_hbm, buf0, buf1, te_v, pacc, sem0,
                   sem1):
    wid = lax.axis_index("s") * 2 + lax.axis_index("c")
    row0 = S1 + wid * NR

    pltpu.sync_copy(te_hbm, te_v)

    def tqbody(j, _):
        for b in range(B):
            te_v[b, pl.ds(j * L, L)] = _bf16_rne(te_v[b, pl.ds(j * L, L)])
        return 0
    lax.fori_loop(0, D // L, tqbody, 0, unroll=8)

    def start(g, buf, sem):
        pltpu.async_copy(x_hbm.at[pl.ds(row0 + g * 2, 2)], buf, sem)

    def wait(g, buf, sem):
        pltpu.make_async_copy(x_hbm.at[pl.ds(row0 + g * 2, 2)], buf,
                              sem).wait()

    def compute(buf, g):
        i0 = g * 2
        for b in range(B):
            zero = jnp.zeros((L,), jnp.float32)

            def jbody(j, accs):
                a0, a1 = accs
                tev = te_v[b, pl.ds(j * L, L)]
                a0 = a0 + _bf16_rne(buf[0, b, pl.ds(j * L, L)]) * tev
                a1 = a1 + _bf16_rne(buf[1, b, pl.ds(j * L, L)]) * tev
                return a0, a1
            a0, a1 = lax.fori_loop(0, D // L, jbody, (zero, zero),
                                   unroll=8)
            pacc[b, i0, :] = a0
            pacc[b, i0 + 1, :] = a1

    start(0, buf0, sem0)

    def qbody(q, _):
        g0 = 2 * q
        start(g0 + 1, buf1, sem1)
        wait(g0, buf0, sem0)
        compute(buf0, g0)

        @pl.when(q + 1 < NG // 2)
        def _():
            start(g0 + 2, buf0, sem0)
        wait(g0 + 1, buf1, sem1)
        compute(buf1, g0 + 1)
        return 0
    lax.fori_loop(0, NG // 2, qbody, 0)

    for b in range(B):
        pltpu.sync_copy(pacc.at[b, pl.ds(0, NR)],
                        w2p_hbm.at[b, pl.ds(wid * NR, NR)])


def _tc_dense_body(x_ref, te_ref, w1_ref):
    i = pl.program_id(0)
    xb = x_ref[...].astype(jnp.bfloat16).astype(jnp.float32)
    te = te_ref[...].astype(jnp.bfloat16).astype(jnp.float32)
    part = jnp.sum(xb * te[None, :, :], axis=-1)
    w1_ref[:, pl.ds(i * SBLK, SBLK)] = part.T


def _tc_final_body(w1_ref, w2p_ref, g_ref, out_ref):
    w2 = jnp.sum(w2p_ref[...], axis=-1)
    w = jnp.concatenate([w1_ref[...], w2], axis=1)
    g = g_ref[...]
    mx = jnp.max(w, axis=1, keepdims=True)
    t = mx - w
    tmx = jnp.max(t, axis=1, keepdims=True)
    p = jnp.exp(t - tmx)
    p_inv = p / jnp.sum(p, axis=1, keepdims=True)
    sc = jnp.log(p_inv + 1e-20) + g
    ku = _sortable_u32(sc)

    prefix = jnp.zeros((B, 1), jnp.uint32)
    for bit in range(31, -1, -1):
        cand = prefix | jnp.uint32(1 << bit)
        cnt = jnp.sum((ku >= cand).astype(jnp.int32), axis=1, keepdims=True)
        prefix = jnp.where(cnt >= N, cand, prefix)
    thr = prefix

    gt = ku > thr
    eq = ku == thr
    r = N - jnp.sum(gt.astype(jnp.int32), axis=1, keepdims=True)
    idx = jax.lax.broadcasted_iota(jnp.int32, (B, S), 1)
    lo = jnp.zeros((B, 1), jnp.int32)
    hi = jnp.full((B, 1), S - 1, jnp.int32)
    for _ in range(12):
        mid = (lo + hi) >> 1
        cnt = jnp.sum((eq & (idx <= mid)).astype(jnp.int32), axis=1,
                      keepdims=True)
        take = cnt >= r
        hi = jnp.where(take, mid, hi)
        lo = jnp.where(take, lo, mid + 1)
    mask = gt | (eq & (idx <= hi))

    neg = jnp.float32(-jnp.inf)
    m2 = jnp.max(jnp.where(mask, neg, w), axis=1, keepdims=True)
    e = jnp.where(mask, 0.0, jnp.exp(w - m2))
    out = e / jnp.sum(e, axis=1, keepdims=True)
    out_ref[...] = out.T


@jax.jit
def kernel(x, te):
    te2 = te[..., 0]
    g = _gumbel_table()

    sc_dense = functools.partial(
        pl.kernel,
        mesh=plsc.VectorSubcoreMesh(core_axis_name="c", subcore_axis_name="s"),
        out_type=jax.ShapeDtypeStruct((B, RS, L), jnp.float32),
        scratch_types=[
            pltpu.VMEM((2, B, D), jnp.float32),
            pltpu.VMEM((2, B, D), jnp.float32),
            pltpu.VMEM((B, D), jnp.float32),
            pltpu.VMEM((B, NR, L), jnp.float32),
            pltpu.SemaphoreType.DMA,
            pltpu.SemaphoreType.DMA,
        ],
        compiler_params=pltpu.CompilerParams(needs_layout_passes=False),
    )(_sc_dense_body)
    w2p = sc_dense(x, te2)

    w1 = pl.pallas_call(
        _tc_dense_body,
        grid=(GRID,),
        in_specs=[
            pl.BlockSpec((SBLK, B, D), lambda i: (i, 0, 0)),
            pl.BlockSpec((B, D), lambda i: (0, 0)),
        ],
        out_specs=pl.BlockSpec((B, S1), lambda i: (0, 0)),
        out_shape=jax.ShapeDtypeStruct((B, S1), jnp.float32),
        compiler_params=pltpu.CompilerParams(
            dimension_semantics=("arbitrary",),
        ),
    )(x, te2)

    out = pl.pallas_call(
        _tc_final_body,
        out_shape=jax.ShapeDtypeStruct((S, B), jnp.float32),
    )(w1, w2p, g)
    return out[..., None]
